# baseline (speedup 1.0000x reference)
import functools

import jax
import jax.numpy as jnp
from jax import lax
from jax.experimental import pallas as pl
from jax.experimental.pallas import tpu as pltpu
from jax.experimental.pallas import tpu_sc as plsc

NCELLS = 200000
NNODES = 100000
FEAT = 128
NW = 32
CHUNK = 100
NCHUNKS = NCELLS // CHUNK
ROW = 393
ITERS = -(-NCHUNKS // NW)
NGROUP = -(-CHUNK // 16)


def _sc_body(u_hbm, t_hbm, cc_hbm, vp_hbm, tri_hbm, out_hbm,
             idx_v, ga_v, gb_v, gc_v, cc_v, vp_v, row_v, t_v, sem):
    wid = lax.axis_index("c") * 16 + lax.axis_index("s")
    lane = jax.lax.iota(jnp.int32, 16)

    pltpu.sync_copy(t_hbm, t_v)
    tvec = t_v[...]
    zero16 = jnp.zeros((16,), dtype=jnp.int32)
    for g in range(NGROUP):
        k = lane + 16 * g
        plsc.store_scatter(row_v, [k, zero16], tvec, mask=k < CHUNK)

    def body(i, carry):
        chunk = wid + i * NW

        @pl.when(chunk < NCHUNKS)
        def _():
            base = chunk * CHUNK
            cps = []
            for v, g_v in ((0, ga_v), (1, gb_v), (2, gc_v)):
                pltpu.sync_copy(tri_hbm.at[v, chunk], idx_v)
                cp = pltpu.async_copy(u_hbm.at[idx_v], g_v, sem)
                cp.wait()
            pltpu.sync_copy(cc_hbm.at[chunk], cc_v)
            pltpu.sync_copy(vp_hbm.at[chunk], vp_v)
            for g in range(NGROUP):
                k = lane + 16 * g
                m = k < CHUNK
                for r in range(2):
                    val = plsc.load_gather(cc_v, [k, zero16 + r], mask=m)
                    plsc.store_scatter(row_v, [k, zero16 + (1 + r)], val,
                                       mask=m)
                for r in range(6):
                    val = plsc.load_gather(vp_v, [k, zero16 + r], mask=m)
                    plsc.store_scatter(row_v, [k, zero16 + (3 + r)], val,
                                       mask=m)

            def cell(c, cc_):
                for v, g_v in ((0, ga_v), (1, gb_v), (2, gc_v)):
                    for kk in range(FEAT // 16):
                        row_v[c, pl.ds(9 + FEAT * v + 16 * kk, 16)] = (
                            g_v[c, pl.ds(16 * kk, 16)])
                return cc_

            lax.fori_loop(0, CHUNK, cell, 0)
            pltpu.sync_copy(row_v, out_hbm.at[pl.ds(base, CHUNK)])

        return carry

    lax.fori_loop(0, ITERS, body, 0)


@functools.partial(jax.jit, static_argnames=())
def kernel(u, t, cell_centers, cell_local_vertex_pos, triangulation):
    u2 = u.reshape(NNODES, FEAT)
    t16 = jnp.broadcast_to(t.reshape(1), (16,)).astype(jnp.float32)
    tri_t = triangulation.astype(jnp.int32).T.reshape(3, NCHUNKS, CHUNK)
    cc3 = cell_centers.reshape(NCHUNKS, CHUNK, 2)
    vp3 = cell_local_vertex_pos.reshape(NCHUNKS, CHUNK, 6)

    mesh = plsc.VectorSubcoreMesh(core_axis_name="c", subcore_axis_name="s")
    out = pl.kernel(
        _sc_body,
        mesh=mesh,
        compiler_params=pltpu.CompilerParams(use_tc_tiling_on_sc=False,
                                             needs_layout_passes=False),
        out_type=jax.ShapeDtypeStruct((NCELLS, ROW), jnp.float32),
        scratch_types=[
            pltpu.VMEM((CHUNK,), jnp.int32),
            pltpu.VMEM((CHUNK, FEAT), jnp.float32),
            pltpu.VMEM((CHUNK, FEAT), jnp.float32),
            pltpu.VMEM((CHUNK, FEAT), jnp.float32),
            pltpu.VMEM((CHUNK, 2), jnp.float32),
            pltpu.VMEM((CHUNK, 6), jnp.float32),
            pltpu.VMEM((CHUNK, ROW), jnp.float32),
            pltpu.VMEM((16,), jnp.float32),
            pltpu.SemaphoreType.DMA,
        ],
    )(u2, t16, cc3, vp3, tri_t)
    return out[None]


# speedup vs baseline: 1.0606x; 1.0606x over previous
import functools

import jax
import jax.numpy as jnp
from jax import lax
from jax.experimental import pallas as pl
from jax.experimental.pallas import tpu as pltpu
from jax.experimental.pallas import tpu_sc as plsc

NCELLS = 200000
NNODES = 100000
FEAT = 128
NW = 32
CHUNK = 50
NCHUNKS = NCELLS // CHUNK
ITERS = NCHUNKS // NW
ROW = 393
NGROUP = -(-CHUNK // 16)

IDX_BYTES = 3 * CHUNK * 4
CC_BYTES = CHUNK * 2 * 4
VP_BYTES = CHUNK * 6 * 4
META_BYTES = IDX_BYTES + CC_BYTES + VP_BYTES
GATHER_BYTES = 3 * CHUNK * FEAT * 4
ROW_BYTES = CHUNK * ROW * 4


def _sc_body(u_hbm, t_hbm, cc_hbm, vp_hbm, tri_hbm, out_hbm,
             idx_v, g_v, cc_v, vp_v, row_v, t_v, isem, gsem, wsem):
    wid = lax.axis_index("c") * 16 + lax.axis_index("s")
    chunk0 = wid * ITERS
    lane = jax.lax.iota(jnp.int32, 16)

    pltpu.sync_copy(t_hbm, t_v)
    tvec = t_v[...]
    zero16 = jnp.zeros((16,), dtype=jnp.int32)
    for g in range(NGROUP):
        k = lane + 16 * g
        plsc.store_scatter(row_v, [k, zero16], tvec, mask=k < CHUNK)

    def load_meta(chunk, b):
        pltpu.async_copy(tri_hbm.at[chunk], idx_v.at[b], isem)
        pltpu.async_copy(cc_hbm.at[chunk], cc_v.at[b], isem)
        pltpu.async_copy(vp_hbm.at[chunk], vp_v.at[b], isem)

    def drain_meta(b):
        pltpu.make_async_copy(tri_hbm.at[0], idx_v.at[b], isem).wait()
        pltpu.make_async_copy(cc_hbm.at[0], cc_v.at[b], isem).wait()
        pltpu.make_async_copy(vp_hbm.at[0], vp_v.at[b], isem).wait()

    def start_gathers(b):
        for j in range(3):
            pltpu.async_copy(u_hbm.at[idx_v.at[b, j]],
                             g_v.at[b, pl.ds(CHUNK * j, CHUNK)], gsem)

    def drain_gathers(b):
        for j in range(3):
            pltpu.make_async_copy(u_hbm.at[pl.ds(0, CHUNK)],
                                  g_v.at[b, pl.ds(CHUNK * j, CHUNK)],
                                  gsem).wait()

    def drain_write():
        pltpu.make_async_copy(out_hbm.at[pl.ds(0, CHUNK)],
                              row_v, wsem).wait()

    load_meta(chunk0, 0)
    drain_meta(0)
    start_gathers(0)
    load_meta(chunk0 + 1, 1)

    def body(i, carry):
        b = lax.rem(i, 2)
        bn = lax.rem(i + 1, 2)
        chunk = chunk0 + i
        base = chunk * CHUNK

        drain_gathers(b)

        @pl.when(i + 1 < ITERS)
        def _():
            drain_meta(bn)
            start_gathers(bn)

        @pl.when(i > 0)
        def _():
            drain_write()

        for g in range(NGROUP):
            k = lane + 16 * g
            m = k < CHUNK
            bk = jnp.broadcast_to(b, (16,))
            for r in range(2):
                val = plsc.load_gather(cc_v, [bk, k, zero16 + r], mask=m)
                plsc.store_scatter(row_v, [k, zero16 + (1 + r)], val, mask=m)
            for r in range(6):
                val = plsc.load_gather(vp_v, [bk, k, zero16 + r], mask=m)
                plsc.store_scatter(row_v, [k, zero16 + (3 + r)], val, mask=m)

        def cell(c, cc_):
            for v in range(3):
                for kk in range(FEAT // 16):
                    row_v[c, pl.ds(9 + FEAT * v + 16 * kk, 16)] = (
                        g_v[b, 3 * c + v, pl.ds(16 * kk, 16)])
            return cc_

        lax.fori_loop(0, CHUNK, cell, 0)

        pltpu.async_copy(row_v, out_hbm.at[pl.ds(base, CHUNK)], wsem)

        @pl.when(i + 2 < ITERS)
        def _():
            load_meta(chunk + 2, b)

        return carry

    lax.fori_loop(0, ITERS, body, 0)
    drain_write()


@functools.partial(jax.jit, static_argnames=())
def kernel(u, t, cell_centers, cell_local_vertex_pos, triangulation):
    u2 = u.reshape(NNODES, FEAT)
    t16 = jnp.broadcast_to(t.reshape(1), (16,)).astype(jnp.float32)
    tri3 = triangulation.astype(jnp.int32).reshape(NCHUNKS, 3, CHUNK)
    cc3 = cell_centers.reshape(NCHUNKS, CHUNK, 2)
    vp3 = cell_local_vertex_pos.reshape(NCHUNKS, CHUNK, 6)

    mesh = plsc.VectorSubcoreMesh(core_axis_name="c", subcore_axis_name="s")
    out = pl.kernel(
        _sc_body,
        mesh=mesh,
        compiler_params=pltpu.CompilerParams(use_tc_tiling_on_sc=False,
                                             needs_layout_passes=False),
        out_type=jax.ShapeDtypeStruct((NCELLS, ROW), jnp.float32),
        scratch_types=[
            pltpu.VMEM((2, 3, CHUNK), jnp.int32),
            pltpu.VMEM((2, 3 * CHUNK, FEAT), jnp.float32),
            pltpu.VMEM((2, CHUNK, 2), jnp.float32),
            pltpu.VMEM((2, CHUNK, 6), jnp.float32),
            pltpu.VMEM((CHUNK, ROW), jnp.float32),
            pltpu.VMEM((16,), jnp.float32),
            pltpu.SemaphoreType.DMA,
            pltpu.SemaphoreType.DMA,
            pltpu.SemaphoreType.DMA,
        ],
    )(u2, t16, cc3, vp3, tri3)
    return out[None]
